# baseline (speedup 1.0000x reference)
import jax
import jax.numpy as jnp
from jax.experimental import pallas as pl
from jax.experimental.pallas import tpu as pltpu

_BITS = 8
_LEVELS = float(2 ** _BITS - 1)
_HALF = float(2 ** (_BITS - 1))

_NB = 32
_K = 9


def _body(x_ref, q_ref, xc_ref, ms_ref, keep_ref, inv_ref):
    p = pl.program_id(0)
    j = pl.program_id(1)

    @pl.when(p == 0)
    def _phase_minmax():
        bmn = jnp.min(x_ref[...])
        bmx = jnp.max(x_ref[...])

        @pl.when(j == 0)
        def _init():
            ms_ref[0] = bmn
            ms_ref[1] = bmx

        @pl.when(j != 0)
        def _acc():
            ms_ref[0] = jnp.minimum(ms_ref[0], bmn)
            ms_ref[1] = jnp.maximum(ms_ref[1], bmx)

        for kk in range(_K):
            @pl.when(j == kk + 1)
            def _stash(kk=kk):
                keep_ref[kk] = x_ref[...]

    @pl.when(p == 1)
    def _phase_quant():
        @pl.when(j == 0)
        def _finalize():
            step = (ms_ref[1] - ms_ref[0]) / _LEVELS
            ms_ref[1] = step
            inv_ref[0] = 1.0 / step

        def _emit(x):
            q_ref[...] = jnp.round(
                (x - ms_ref[0]) * inv_ref[0] - _HALF
            ).astype(jnp.int8)
            xc_ref[...] = x

        @pl.when((j == 0) | (j > _K))
        def _from_hbm():
            _emit(x_ref[...])

        for kk in range(_K):
            @pl.when(j == kk + 1)
            def _from_keep(kk=kk):
                _emit(keep_ref[kk])


def kernel(input):
    shape = input.shape
    C = shape[-1]
    R = 1
    for s in shape[:-1]:
        R *= s
    x = input.reshape(R, C)

    nb = _NB
    bs = R // nb

    def _in_map(p, j):
        return (jnp.where((p == 1) & (j <= _K), 0, j), 0)

    q, xc, ms = pl.pallas_call(
        _body,
        grid=(2, nb),
        in_specs=[pl.BlockSpec((bs, C), _in_map)],
        out_specs=[
            pl.BlockSpec((bs, C), lambda p, j: (jnp.where(p == 0, 0, j), 0)),
            pl.BlockSpec((bs, C), lambda p, j: (jnp.where(p == 0, 0, j), 0)),
            pl.BlockSpec(memory_space=pltpu.SMEM),
        ],
        out_shape=[
            jax.ShapeDtypeStruct((R, C), jnp.int8),
            jax.ShapeDtypeStruct((R, C), jnp.float32),
            jax.ShapeDtypeStruct((2,), jnp.float32),
        ],
        scratch_shapes=[
            pltpu.VMEM((_K, bs, C), jnp.float32),
            pltpu.SMEM((1,), jnp.float32),
        ],
        compiler_params=pltpu.CompilerParams(
            dimension_semantics=("arbitrary", "arbitrary"),
        ),
    )(x)

    return (xc.reshape(shape), q.reshape(shape), ms)


# speedup vs baseline: 1.0578x; 1.0578x over previous
import jax
import jax.numpy as jnp
from jax.experimental import pallas as pl
from jax.experimental.pallas import tpu as pltpu

_BITS = 8
_LEVELS = float(2 ** _BITS - 1)
_HALF = float(2 ** (_BITS - 1))

_NB = 16
_K = 2


def _body(x_ref, q_ref, xc_ref, ms_ref, keep_ref, inv_ref):
    p = pl.program_id(0)
    j = pl.program_id(1)

    @pl.when(p == 0)
    def _phase_minmax():
        bmn = jnp.min(x_ref[...])
        bmx = jnp.max(x_ref[...])

        @pl.when(j == 0)
        def _init():
            ms_ref[0] = bmn
            ms_ref[1] = bmx

        @pl.when(j != 0)
        def _acc():
            ms_ref[0] = jnp.minimum(ms_ref[0], bmn)
            ms_ref[1] = jnp.maximum(ms_ref[1], bmx)

        xc_ref[...] = x_ref[...]

        for kk in range(_K):
            @pl.when(j == kk + 1)
            def _stash(kk=kk):
                keep_ref[kk] = x_ref[...]

    @pl.when(p == 1)
    def _phase_quant():
        @pl.when(j == 0)
        def _finalize():
            step = (ms_ref[1] - ms_ref[0]) / _LEVELS
            ms_ref[1] = step
            inv_ref[0] = 1.0 / step

        def _emit(x):
            q_ref[...] = jnp.round(
                (x - ms_ref[0]) * inv_ref[0] - _HALF
            ).astype(jnp.int8)

        @pl.when((j == 0) | (j > _K))
        def _from_hbm():
            _emit(x_ref[...])

        for kk in range(_K):
            @pl.when(j == kk + 1)
            def _from_keep(kk=kk):
                _emit(keep_ref[kk])


def kernel(input):
    shape = input.shape
    C = shape[-1]
    R = 1
    for s in shape[:-1]:
        R *= s
    x = input.reshape(R, C)

    nb = _NB
    bs = R // nb

    def _in_map(p, j):
        return (jnp.where((p == 1) & (j <= _K), 0, j), 0)

    q, xc, ms = pl.pallas_call(
        _body,
        grid=(2, nb),
        in_specs=[pl.BlockSpec((bs, C), _in_map)],
        out_specs=[
            pl.BlockSpec((bs, C), lambda p, j: (jnp.where(p == 0, 0, j), 0)),
            pl.BlockSpec((bs, C), lambda p, j: (jnp.where(p == 0, j, nb - 1), 0)),
            pl.BlockSpec(memory_space=pltpu.SMEM),
        ],
        out_shape=[
            jax.ShapeDtypeStruct((R, C), jnp.int8),
            jax.ShapeDtypeStruct((R, C), jnp.float32),
            jax.ShapeDtypeStruct((2,), jnp.float32),
        ],
        scratch_shapes=[
            pltpu.VMEM((_K, bs, C), jnp.float32),
            pltpu.SMEM((1,), jnp.float32),
        ],
        compiler_params=pltpu.CompilerParams(
            dimension_semantics=("arbitrary", "arbitrary"),
        ),
    )(x)

    return (xc.reshape(shape), q.reshape(shape), ms)


# speedup vs baseline: 1.0737x; 1.0150x over previous
import jax
import jax.numpy as jnp
from jax.experimental import pallas as pl
from jax.experimental.pallas import tpu as pltpu

_BITS = 8
_LEVELS = float(2 ** _BITS - 1)
_HALF = float(2 ** (_BITS - 1))

_NB = 16
_K = 3


def _body(x_ref, q_ref, xc_ref, ms_ref, keep_ref, inv_ref):
    p = pl.program_id(0)
    j = pl.program_id(1)

    @pl.when(p == 0)
    def _phase_minmax():
        bmn = jnp.min(x_ref[...])
        bmx = jnp.max(x_ref[...])

        @pl.when(j == 0)
        def _init():
            ms_ref[0] = bmn
            ms_ref[1] = bmx

        @pl.when(j != 0)
        def _acc():
            ms_ref[0] = jnp.minimum(ms_ref[0], bmn)
            ms_ref[1] = jnp.maximum(ms_ref[1], bmx)

        xc_ref[...] = x_ref[...]

        for kk in range(_K):
            @pl.when(j == kk + 1)
            def _stash(kk=kk):
                keep_ref[kk] = x_ref[...]

    @pl.when(p == 1)
    def _phase_quant():
        @pl.when(j == 0)
        def _finalize():
            step = (ms_ref[1] - ms_ref[0]) / _LEVELS
            ms_ref[1] = step
            inv_ref[0] = 1.0 / step

        def _emit(x):
            q_ref[...] = jnp.round(
                (x - ms_ref[0]) * inv_ref[0] - _HALF
            ).astype(jnp.int8)

        @pl.when((j == 0) | (j > _K))
        def _from_hbm():
            _emit(x_ref[...])

        for kk in range(_K):
            @pl.when(j == kk + 1)
            def _from_keep(kk=kk):
                _emit(keep_ref[kk])


def kernel(input):
    shape = input.shape
    C = shape[-1]
    R = 1
    for s in shape[:-1]:
        R *= s
    x = input.reshape(R, C)

    nb = _NB
    bs = R // nb

    def _in_map(p, j):
        return (jnp.where((p == 1) & (j <= _K), 0, j), 0)

    q, xc, ms = pl.pallas_call(
        _body,
        grid=(2, nb),
        in_specs=[pl.BlockSpec((bs, C), _in_map)],
        out_specs=[
            pl.BlockSpec((bs, C), lambda p, j: (jnp.where(p == 0, 0, j), 0)),
            pl.BlockSpec((bs, C), lambda p, j: (jnp.where(p == 0, j, nb - 1), 0)),
            pl.BlockSpec(memory_space=pltpu.SMEM),
        ],
        out_shape=[
            jax.ShapeDtypeStruct((R, C), jnp.int8),
            jax.ShapeDtypeStruct((R, C), jnp.float32),
            jax.ShapeDtypeStruct((2,), jnp.float32),
        ],
        scratch_shapes=[
            pltpu.VMEM((_K, bs, C), jnp.float32),
            pltpu.SMEM((1,), jnp.float32),
        ],
        compiler_params=pltpu.CompilerParams(
            dimension_semantics=("arbitrary", "arbitrary"),
            vmem_limit_bytes=128 * 1024 * 1024,
        ),
    )(x)

    return (xc.reshape(shape), q.reshape(shape), ms)


# speedup vs baseline: 1.0859x; 1.0114x over previous
import jax
import jax.numpy as jnp
from jax.experimental import pallas as pl
from jax.experimental.pallas import tpu as pltpu

_BITS = 8
_LEVELS = float(2 ** _BITS - 1)
_HALF = float(2 ** (_BITS - 1))

_NB = 16
_K = 3


def _body(x_ref, q_ref, xc_ref, ms_ref, keep_ref, inv_ref):
    p = pl.program_id(0)
    j = pl.program_id(1)

    @pl.when(p == 0)
    def _phase_minmax():
        bmn = jnp.min(x_ref[...])
        bmx = jnp.max(x_ref[...])

        @pl.when(j == 0)
        def _init():
            ms_ref[0] = bmn
            ms_ref[1] = bmx

        @pl.when(j != 0)
        def _acc():
            ms_ref[0] = jnp.minimum(ms_ref[0], bmn)
            ms_ref[1] = jnp.maximum(ms_ref[1], bmx)

        xc_ref[...] = x_ref[...]

        for kk in range(_K):
            @pl.when(j == _NB - _K + kk)
            def _stash(kk=kk):
                keep_ref[kk] = x_ref[...]

    @pl.when(p == 1)
    def _phase_quant():
        @pl.when(j == 0)
        def _finalize():
            step = (ms_ref[1] - ms_ref[0]) / _LEVELS
            ms_ref[1] = step
            inv_ref[0] = 1.0 / step

        def _emit(x):
            q_ref[...] = jnp.round(
                (x - ms_ref[0]) * inv_ref[0] - _HALF
            ).astype(jnp.int8)

        @pl.when(j < _NB - _K)
        def _from_hbm():
            _emit(x_ref[...])

        for kk in range(_K):
            @pl.when(j == _NB - _K + kk)
            def _from_keep(kk=kk):
                _emit(keep_ref[kk])


def kernel(input):
    shape = input.shape
    C = shape[-1]
    R = 1
    for s in shape[:-1]:
        R *= s
    x = input.reshape(R, C)

    nb = _NB
    bs = R // nb

    def _in_map(p, j):
        return (jnp.where((p == 1) & (j >= _NB - _K), _NB - _K - 1, j), 0)

    q, xc, ms = pl.pallas_call(
        _body,
        grid=(2, nb),
        in_specs=[pl.BlockSpec((bs, C), _in_map)],
        out_specs=[
            pl.BlockSpec((bs, C), lambda p, j: (jnp.where(p == 0, 0, j), 0)),
            pl.BlockSpec((bs, C), lambda p, j: (jnp.where(p == 0, j, nb - 1), 0)),
            pl.BlockSpec(memory_space=pltpu.SMEM),
        ],
        out_shape=[
            jax.ShapeDtypeStruct((R, C), jnp.int8),
            jax.ShapeDtypeStruct((R, C), jnp.float32),
            jax.ShapeDtypeStruct((2,), jnp.float32),
        ],
        scratch_shapes=[
            pltpu.VMEM((_K, bs, C), jnp.float32),
            pltpu.SMEM((1,), jnp.float32),
        ],
        compiler_params=pltpu.CompilerParams(
            dimension_semantics=("arbitrary", "arbitrary"),
            vmem_limit_bytes=128 * 1024 * 1024,
        ),
    )(x)

    return (xc.reshape(shape), q.reshape(shape), ms)
